# baseline (speedup 1.0000x reference)
import functools

import jax
import jax.numpy as jnp
from jax import lax
from jax.experimental import pallas as pl
from jax.experimental.pallas import tpu as pltpu
from jax.experimental.pallas import tpu_sc as plsc

N = 10000
D = 128
D_OUT = 64
E = 320000

NC = 2
NS = 16
NW = NC * NS

K = 128
CH = 80
EPT = K * CH
EPAD = EPT * NW
NPAD = 10240
RPT = NPAD // NS
GRID2 = 5
BLK2 = N // GRID2


def _mesh():
    return plsc.VectorSubcoreMesh(
        core_axis_name="c", subcore_axis_name="s", num_cores=NC, num_subcores=NS
    )


def _sc_degree(dst2d):
    @functools.partial(
        pl.kernel,
        out_type=jax.ShapeDtypeStruct((NC * NPAD,), jnp.float32),
        mesh=_mesh(),
        scratch_types=[
            pltpu.VMEM((CH, K), jnp.int32),
            pltpu.VMEM((K,), jnp.float32),
            pltpu.VMEM((RPT,), jnp.float32),
            pltpu.VMEM_SHARED((NPAD,), jnp.float32),
            pltpu.SemaphoreType.DMA,
        ],
    )
    def k(dst_hbm, out_hbm, dst_v, ones_v, zbuf, acc, sem):
        c = lax.axis_index("c")
        s = lax.axis_index("s")
        wid = c * NS + s
        r0 = s * RPT

        def fill(i, carry):
            ones_v[pl.ds(i * 16, 16)] = jnp.ones((16,), jnp.float32)
            return carry
        lax.fori_loop(0, K // 16, fill, 0)

        def zfill(i, carry):
            zbuf[pl.ds(i * 16, 16)] = jnp.zeros((16,), jnp.float32)
            return carry
        lax.fori_loop(0, RPT // 16, zfill, 0)
        pltpu.sync_copy(zbuf, acc.at[pl.ds(r0, RPT)])

        pltpu.sync_copy(dst_hbm.at[pl.ds(wid * CH, CH)], dst_v)
        plsc.subcore_barrier()

        def body(g, carry):
            pltpu.sync_copy(ones_v, acc.at[dst_v.at[g]], add=True)
            return carry
        lax.fori_loop(0, CH, body, 0)

        plsc.subcore_barrier()
        pltpu.sync_copy(acc.at[pl.ds(r0, RPT)],
                        out_hbm.at[pl.ds(c * NPAD + r0, RPT)])

    return k(dst2d)


def _sc_scatter(y, src1d, dst1d):
    @functools.partial(
        pl.kernel,
        out_type=jax.ShapeDtypeStruct((NC * NPAD, D), jnp.float32),
        mesh=_mesh(),
        scratch_types=[
            [pltpu.VMEM((K,), jnp.int32)] * 4,
            [pltpu.VMEM((K,), jnp.int32)] * 4,
            pltpu.VMEM((K, D), jnp.float32),
            pltpu.VMEM((K, D), jnp.float32),
            pltpu.VMEM_SHARED((NPAD, D), jnp.float32),
            pltpu.SemaphoreType.DMA,
            pltpu.SemaphoreType.DMA,
            pltpu.SemaphoreType.DMA,
            pltpu.SemaphoreType.DMA,
            pltpu.SemaphoreType.DMA,
            pltpu.SemaphoreType.DMA,
        ],
    )
    def k(y_hbm, src_hbm, dst_hbm, out_hbm, sb, db, rows0, rows1, acc,
          gs0, gs1, is0, is1, is2, is3):
        c = lax.axis_index("c")
        s = lax.axis_index("s")
        wid = c * NS + s
        r0 = s * RPT
        rows = (rows0, rows1)
        gsem = (gs0, gs1)
        isem = (is0, is1, is2, is3)
        e0 = wid * EPT

        def load_idx(j, off):
            pltpu.async_copy(src_hbm.at[pl.ds(off, K)], sb[j], isem[j])
            pltpu.async_copy(dst_hbm.at[pl.ds(off, K)], db[j], isem[j])

        def wait_idx(j):
            pltpu.make_async_copy(src_hbm.at[pl.ds(0, K)], sb[j], isem[j]).wait()
            pltpu.make_async_copy(dst_hbm.at[pl.ds(0, K)], db[j], isem[j]).wait()

        for j in range(4):
            load_idx(j, e0 + j * K)
        def zfill(i, carry):
            for v in range(D // 16):
                rows1[i, pl.ds(v * 16, 16)] = jnp.zeros((16,), jnp.float32)
            return carry
        lax.fori_loop(0, K, zfill, 0)
        for i in range(RPT // K):
            pltpu.sync_copy(rows1, acc.at[pl.ds(r0 + i * K, K)])
        for b in range(2):
            wait_idx(b)
            pltpu.async_copy(y_hbm.at[sb[b]], rows[b], gsem[b])
        plsc.subcore_barrier()

        def outer(t, carry):
            for j in range(4):
                g = 4 * t + j
                b = j % 2
                j2 = (j + 2) % 4
                pltpu.make_async_copy(
                    y_hbm.at[sb[j]], rows[b], gsem[b]).wait()
                wait_idx(j2)
                pltpu.sync_copy(rows[b], acc.at[db[j]], add=True)
                g4 = jnp.where(g + 4 < CH, g + 4, 0)
                load_idx(j, e0 + g4 * K)
                pltpu.async_copy(y_hbm.at[sb[j2]], rows[b], gsem[b])
            return carry
        lax.fori_loop(0, CH // 4, outer, 0)

        pltpu.make_async_copy(y_hbm.at[sb[0]], rows0, gs0).wait()
        pltpu.make_async_copy(y_hbm.at[sb[0]], rows1, gs1).wait()
        wait_idx(2)
        wait_idx(3)

        plsc.subcore_barrier()
        pltpu.sync_copy(acc.at[pl.ds(r0, RPT)],
                        out_hbm.at[pl.ds(c * NPAD + r0, RPT)])

    return k(y, src1d, dst1d)


def _tc_scale_mm(x, W, degp):
    def body(x_ref, w_ref, d0_ref, d1_ref, o_ref):
        dinv = lax.rsqrt(d0_ref[0] + d1_ref[0] + 1.0)
        xw = jnp.dot(x_ref[...], w_ref[...], preferred_element_type=jnp.float32)
        o_ref[...] = xw * dinv

    return pl.pallas_call(
        body,
        grid=(GRID2,),
        in_specs=[
            pl.BlockSpec((BLK2, D), lambda i: (i, 0)),
            pl.BlockSpec((D, D), lambda i: (0, 0)),
            pl.BlockSpec((1, BLK2, 1), lambda i: (0, i, 0)),
            pl.BlockSpec((1, BLK2, 1), lambda i: (1, i, 0)),
        ],
        out_specs=pl.BlockSpec((BLK2, D), lambda i: (i, 0)),
        out_shape=jax.ShapeDtypeStruct((N, D), jnp.float32),
    )(x, W, degp, degp)


def _tc_mid(p, y, degp, b, W):
    def body(p0_ref, p1_ref, y_ref, d0_ref, d1_ref, b_ref, w_ref, o_ref):
        dinv = lax.rsqrt(d0_ref[0] + d1_ref[0] + 1.0)
        h = (p0_ref[0] + p1_ref[0] + y_ref[...]) * dinv + b_ref[...]
        h = jnp.maximum(h, 0.0)
        o_ref[...] = jnp.dot(h, w_ref[...], preferred_element_type=jnp.float32) * dinv

    return pl.pallas_call(
        body,
        grid=(GRID2,),
        in_specs=[
            pl.BlockSpec((1, BLK2, D), lambda i: (0, i, 0)),
            pl.BlockSpec((1, BLK2, D), lambda i: (1, i, 0)),
            pl.BlockSpec((BLK2, D), lambda i: (i, 0)),
            pl.BlockSpec((1, BLK2, 1), lambda i: (0, i, 0)),
            pl.BlockSpec((1, BLK2, 1), lambda i: (1, i, 0)),
            pl.BlockSpec((1, D), lambda i: (0, 0)),
            pl.BlockSpec((D, D), lambda i: (0, 0)),
        ],
        out_specs=pl.BlockSpec((BLK2, D), lambda i: (i, 0)),
        out_shape=jax.ShapeDtypeStruct((N, D), jnp.float32),
    )(p, p, y, degp, degp, b, W)


def _tc_final(p, y, degp, b, Wfc, bfc):
    def body(p0_ref, p1_ref, y_ref, d0_ref, d1_ref, b_ref, w_ref, bfc_ref, o_ref):
        dinv = lax.rsqrt(d0_ref[0] + d1_ref[0] + 1.0)
        h = (p0_ref[0] + p1_ref[0] + y_ref[...]) * dinv + b_ref[...]
        h = jnp.maximum(h, 0.0)
        o_ref[...] = (
            jnp.dot(h, w_ref[...], preferred_element_type=jnp.float32) + bfc_ref[...]
        )

    return pl.pallas_call(
        body,
        grid=(GRID2,),
        in_specs=[
            pl.BlockSpec((1, BLK2, D), lambda i: (0, i, 0)),
            pl.BlockSpec((1, BLK2, D), lambda i: (1, i, 0)),
            pl.BlockSpec((BLK2, D), lambda i: (i, 0)),
            pl.BlockSpec((1, BLK2, 1), lambda i: (0, i, 0)),
            pl.BlockSpec((1, BLK2, 1), lambda i: (1, i, 0)),
            pl.BlockSpec((1, D), lambda i: (0, 0)),
            pl.BlockSpec((D, D_OUT), lambda i: (0, 0)),
            pl.BlockSpec((1, D_OUT), lambda i: (0, 0)),
        ],
        out_specs=pl.BlockSpec((BLK2, D_OUT), lambda i: (i, 0)),
        out_shape=jax.ShapeDtypeStruct((N, D_OUT), jnp.float32),
    )(p, p, y, degp, degp, b, Wfc, bfc)


def kernel(x, edge_index, W1, b1, W2, b2, Wfc, bfc):
    src = edge_index[0].astype(jnp.int32)
    dst = edge_index[1].astype(jnp.int32)
    pad = EPAD - E
    pad_src = (jnp.arange(pad, dtype=jnp.int32) * 37) % N
    pad_dst = N + (jnp.arange(pad, dtype=jnp.int32) % (NPAD - N))
    src1d = jnp.concatenate([src, pad_src])
    dst1d = jnp.concatenate([dst, pad_dst])
    dst2d = dst1d.reshape(EPAD // K, K)

    degp = _sc_degree(dst2d).reshape(2, NPAD, 1)
    y1 = _tc_scale_mm(x, W1, degp)
    p1 = _sc_scatter(y1, src1d, dst1d).reshape(2, NPAD, D)
    y2 = _tc_mid(p1, y1, degp, b1.reshape(1, D), W2)
    p2 = _sc_scatter(y2, src1d, dst1d).reshape(2, NPAD, D)
    return _tc_final(p2, y2, degp, b2.reshape(1, D), Wfc, bfc.reshape(1, D_OUT))


# speedup vs baseline: 1.0424x; 1.0424x over previous
import functools

import jax
import jax.numpy as jnp
from jax import lax
from jax.experimental import pallas as pl
from jax.experimental.pallas import tpu as pltpu
from jax.experimental.pallas import tpu_sc as plsc

N = 10000
D = 128
D_OUT = 64
E = 320000

NC = 2
NS = 16
NW = NC * NS

K = 128
CH = 80
EPT = K * CH
EPAD = EPT * NW
NPAD = 10240
RPT = NPAD // NS
GRID2 = 5
BLK2 = N // GRID2


def _mesh():
    return plsc.VectorSubcoreMesh(
        core_axis_name="c", subcore_axis_name="s", num_cores=NC, num_subcores=NS
    )


def _sc_degree(dst2d):
    @functools.partial(
        pl.kernel,
        out_type=jax.ShapeDtypeStruct((NC * NPAD,), jnp.float32),
        mesh=_mesh(),
        scratch_types=[
            pltpu.VMEM((CH, K), jnp.int32),
            pltpu.VMEM((K,), jnp.float32),
            pltpu.VMEM((RPT,), jnp.float32),
            pltpu.VMEM_SHARED((NPAD,), jnp.float32),
            pltpu.SemaphoreType.DMA,
        ],
    )
    def k(dst_hbm, out_hbm, dst_v, ones_v, zbuf, acc, sem):
        c = lax.axis_index("c")
        s = lax.axis_index("s")
        wid = c * NS + s
        r0 = s * RPT

        def fill(i, carry):
            ones_v[pl.ds(i * 16, 16)] = jnp.ones((16,), jnp.float32)
            return carry
        lax.fori_loop(0, K // 16, fill, 0)

        def zfill(i, carry):
            zbuf[pl.ds(i * 16, 16)] = jnp.zeros((16,), jnp.float32)
            return carry
        lax.fori_loop(0, RPT // 16, zfill, 0)
        pltpu.sync_copy(zbuf, acc.at[pl.ds(r0, RPT)])

        pltpu.sync_copy(dst_hbm.at[pl.ds(wid * CH, CH)], dst_v)
        plsc.subcore_barrier()

        def body(g, carry):
            pltpu.sync_copy(ones_v, acc.at[dst_v.at[g]], add=True)
            return carry
        lax.fori_loop(0, CH, body, 0)

        plsc.subcore_barrier()
        pltpu.sync_copy(acc.at[pl.ds(r0, RPT)],
                        out_hbm.at[pl.ds(c * NPAD + r0, RPT)])

    return k(dst2d)


def _sc_scatter(y, src1d, dst1d):
    @functools.partial(
        pl.kernel,
        out_type=jax.ShapeDtypeStruct((NC * NPAD, D), jnp.float32),
        mesh=_mesh(),
        scratch_types=[
            [pltpu.VMEM((K,), jnp.int32)] * 4,
            [pltpu.VMEM((K,), jnp.int32)] * 4,
            pltpu.VMEM((K, D), jnp.float32),
            pltpu.VMEM((K, D), jnp.float32),
            pltpu.VMEM_SHARED((NPAD, D), jnp.float32),
            pltpu.SemaphoreType.DMA,
            pltpu.SemaphoreType.DMA,
            pltpu.SemaphoreType.DMA,
            pltpu.SemaphoreType.DMA,
            pltpu.SemaphoreType.DMA,
            pltpu.SemaphoreType.DMA,
        ],
    )
    def k(y_hbm, src_hbm, dst_hbm, out_hbm, sb, db, rows0, rows1, acc,
          gs0, gs1, is0, is1, is2, is3):
        c = lax.axis_index("c")
        s = lax.axis_index("s")
        wid = c * NS + s
        r0 = s * RPT
        rows = (rows0, rows1)
        gsem = (gs0, gs1)
        isem = (is0, is1, is2, is3)
        e0 = wid * EPT

        def load_idx(j, off):
            pltpu.async_copy(src_hbm.at[pl.ds(off, K)], sb[j], isem[j])
            pltpu.async_copy(dst_hbm.at[pl.ds(off, K)], db[j], isem[j])

        def wait_idx(j):
            pltpu.make_async_copy(src_hbm.at[pl.ds(0, K)], sb[j], isem[j]).wait()
            pltpu.make_async_copy(dst_hbm.at[pl.ds(0, K)], db[j], isem[j]).wait()

        for j in range(4):
            load_idx(j, e0 + j * K)
        def zfill(i, carry):
            for v in range(D // 16):
                rows1[i, pl.ds(v * 16, 16)] = jnp.zeros((16,), jnp.float32)
            return carry
        lax.fori_loop(0, K, zfill, 0)
        for i in range(RPT // K):
            pltpu.sync_copy(rows1, acc.at[pl.ds(r0 + i * K, K)])
        for b in range(2):
            wait_idx(b)
            pltpu.async_copy(y_hbm.at[sb[b]], rows[b], gsem[b])
        plsc.subcore_barrier()

        def outer(t, carry):
            for j in range(4):
                g = 4 * t + j
                b = j % 2
                j2 = (j + 2) % 4
                pltpu.make_async_copy(
                    y_hbm.at[sb[j]], rows[b], gsem[b]).wait()
                wait_idx(j2)
                pltpu.sync_copy(rows[b], acc.at[db[j]], add=True)
                g4 = jnp.where(g + 4 < CH, g + 4, 0)
                load_idx(j, e0 + g4 * K)
                pltpu.async_copy(y_hbm.at[sb[j2]], rows[b], gsem[b])
            return carry
        lax.fori_loop(0, CH // 4, outer, 0)

        pltpu.make_async_copy(y_hbm.at[sb[0]], rows0, gs0).wait()
        pltpu.make_async_copy(y_hbm.at[sb[0]], rows1, gs1).wait()
        wait_idx(2)
        wait_idx(3)

        plsc.subcore_barrier()
        pltpu.sync_copy(acc.at[pl.ds(r0, RPT)],
                        out_hbm.at[pl.ds(c * NPAD + r0, RPT)])

    return k(y, src1d, dst1d)


def _tc_scale_mm(x, W, deg2d):
    def body(x_ref, w_ref, deg_ref, o_ref):
        dinv = lax.rsqrt(deg_ref[...] + 1.0)
        xw = jnp.dot(x_ref[...], w_ref[...], preferred_element_type=jnp.float32)
        o_ref[...] = xw * dinv

    return pl.pallas_call(
        body,
        grid=(GRID2,),
        in_specs=[
            pl.BlockSpec((BLK2, D), lambda i: (i, 0)),
            pl.BlockSpec((D, D), lambda i: (0, 0)),
            pl.BlockSpec((BLK2, 1), lambda i: (i, 0)),
        ],
        out_specs=pl.BlockSpec((BLK2, D), lambda i: (i, 0)),
        out_shape=jax.ShapeDtypeStruct((N, D), jnp.float32),
    )(x, W, deg2d)


def _tc_mid(p, y, deg2d, b, W):
    def body(p0_ref, p1_ref, y_ref, deg_ref, b_ref, w_ref, o_ref):
        dinv = lax.rsqrt(deg_ref[...] + 1.0)
        h = (p0_ref[0] + p1_ref[0] + y_ref[...]) * dinv + b_ref[...]
        h = jnp.maximum(h, 0.0)
        o_ref[...] = jnp.dot(h, w_ref[...], preferred_element_type=jnp.float32) * dinv

    return pl.pallas_call(
        body,
        grid=(GRID2,),
        in_specs=[
            pl.BlockSpec((1, BLK2, D), lambda i: (0, i, 0)),
            pl.BlockSpec((1, BLK2, D), lambda i: (1, i, 0)),
            pl.BlockSpec((BLK2, D), lambda i: (i, 0)),
            pl.BlockSpec((BLK2, 1), lambda i: (i, 0)),
            pl.BlockSpec((1, D), lambda i: (0, 0)),
            pl.BlockSpec((D, D), lambda i: (0, 0)),
        ],
        out_specs=pl.BlockSpec((BLK2, D), lambda i: (i, 0)),
        out_shape=jax.ShapeDtypeStruct((N, D), jnp.float32),
    )(p, p, y, deg2d, b, W)


def _tc_final(p, y, deg2d, b, Wfc, bfc):
    def body(p0_ref, p1_ref, y_ref, deg_ref, b_ref, w_ref, bfc_ref, o_ref):
        dinv = lax.rsqrt(deg_ref[...] + 1.0)
        h = (p0_ref[0] + p1_ref[0] + y_ref[...]) * dinv + b_ref[...]
        h = jnp.maximum(h, 0.0)
        o_ref[...] = (
            jnp.dot(h, w_ref[...], preferred_element_type=jnp.float32) + bfc_ref[...]
        )

    return pl.pallas_call(
        body,
        grid=(GRID2,),
        in_specs=[
            pl.BlockSpec((1, BLK2, D), lambda i: (0, i, 0)),
            pl.BlockSpec((1, BLK2, D), lambda i: (1, i, 0)),
            pl.BlockSpec((BLK2, D), lambda i: (i, 0)),
            pl.BlockSpec((BLK2, 1), lambda i: (i, 0)),
            pl.BlockSpec((1, D), lambda i: (0, 0)),
            pl.BlockSpec((D, D_OUT), lambda i: (0, 0)),
            pl.BlockSpec((1, D_OUT), lambda i: (0, 0)),
        ],
        out_specs=pl.BlockSpec((BLK2, D_OUT), lambda i: (i, 0)),
        out_shape=jax.ShapeDtypeStruct((N, D_OUT), jnp.float32),
    )(p, p, y, deg2d, b, Wfc, bfc)


def kernel(x, edge_index, W1, b1, W2, b2, Wfc, bfc):
    src = edge_index[0].astype(jnp.int32)
    dst = edge_index[1].astype(jnp.int32)
    pad = EPAD - E
    pad_src = (jnp.arange(pad, dtype=jnp.int32) * 37) % N
    pad_dst = N + (jnp.arange(pad, dtype=jnp.int32) % (NPAD - N))
    src1d = jnp.concatenate([src, pad_src])
    dst1d = jnp.concatenate([dst, pad_dst])
    dst2d = dst1d.reshape(EPAD // K, K)

    degp = _sc_degree(dst2d)
    deg2d = (degp[:NPAD] + degp[NPAD:]).reshape(NPAD, 1)
    y1 = _tc_scale_mm(x, W1, deg2d)
    p1 = _sc_scatter(y1, src1d, dst1d).reshape(2, NPAD, D)
    y2 = _tc_mid(p1, y1, deg2d, b1.reshape(1, D), W2)
    p2 = _sc_scatter(y2, src1d, dst1d).reshape(2, NPAD, D)
    return _tc_final(p2, y2, deg2d, b2.reshape(1, D), Wfc, bfc.reshape(1, D_OUT))
